# baseline (speedup 1.0000x reference)
import functools

import jax
import jax.numpy as jnp
from jax import lax
from jax.experimental import pallas as pl
from jax.experimental.pallas import tpu as pltpu
from jax.experimental.pallas import tpu_sc as plsc

_L = 200
_D = 64

_info = plsc.get_sparse_core_info()
_NC = _info.num_cores
_NS = _info.num_subcores
_NW = _NC * _NS


def _sc_pool(x, tab, lo):
    Lh = x.shape[0]
    B = x.shape[1]
    vh = tab.shape[0]
    bpw = B // _NW
    bpc = bpw * _NS

    mesh = plsc.VectorSubcoreMesh(core_axis_name="c", subcore_axis_name="s")
    nbuf = 8
    nch = Lh // nbuf

    @functools.partial(
        pl.kernel,
        mesh=mesh,
        out_type=jax.ShapeDtypeStruct((B, _D), jnp.float32),
        scratch_types=[
            pltpu.VMEM((Lh, bpw), jnp.int32),
            pltpu.VMEM((nbuf, bpw), jnp.int32),
            pltpu.VMEM((nbuf, bpw), jnp.int32),
            pltpu.VMEM((nbuf * bpw, _D), jnp.float32),
            pltpu.VMEM_SHARED((2 * bpc, _D), jnp.float32),
            pltpu.SemaphoreType.DMA((nbuf,)),
            pltpu.SemaphoreType.DMA((nbuf,)),
        ],
        compiler_params=pltpu.CompilerParams(use_tc_tiling_on_sc=False),
    )
    def pool(x_hbm, tab_hbm, out_hbm, idx_v, gidx_v, slot_v, bufs,
             acc_sh, gsem, ssem):
        cid = lax.axis_index("c")
        sid = lax.axis_index("s")
        base = pl.multiple_of((sid * _NC + cid) * bpw, bpw)
        s2 = pl.multiple_of(2 * sid * bpw, 2 * bpw)

        zvec = jnp.zeros((16,), jnp.float32)
        iot = lax.iota(jnp.int32, 16)

        def zero_row(i, carry):
            for d in range(_D // 16):
                bufs[i, pl.ds(d * 16, 16)] = zvec
            return carry
        lax.fori_loop(0, 2 * bpw, zero_row, 0)
        pltpu.sync_copy(bufs.at[pl.ds(0, 2 * bpw)],
                        acc_sh.at[pl.ds(s2, 2 * bpw)])

        pltpu.sync_copy(x_hbm.at[:, pl.ds(base, bpw)], idx_v)

        def fill_gidx(l, b):
            for k in range(bpw // 16):
                v = idx_v[l, pl.ds(k * 16, 16)] - lo
                inh = (v >= 0) & (v < vh)
                gidx_v[b, pl.ds(k * 16, 16)] = jnp.where(inh, v, 0)

        def fill_slot(l, b):
            for k in range(bpw // 16):
                v = idx_v[l, pl.ds(k * 16, 16)] - lo
                inh = (v >= 0) & (v < vh)
                junk = jnp.where(inh, 0, bpw).astype(jnp.int32)
                slot_v[b, pl.ds(k * 16, 16)] = s2 + (k * 16 + iot) + junk

        def gather(b):
            return pltpu.async_copy(
                tab_hbm.at[gidx_v.at[b]],
                bufs.at[pl.ds(b * bpw, bpw)], gsem.at[b])

        def scatter_add(b):
            return pltpu.async_copy(
                bufs.at[pl.ds(b * bpw, bpw)],
                acc_sh.at[slot_v.at[b]], ssem.at[b], add=True)

        def wait_gather(b):
            pltpu.make_async_copy(
                tab_hbm.at[gidx_v.at[b]],
                bufs.at[pl.ds(b * bpw, bpw)], gsem.at[b]).wait()

        def wait_scatter(b):
            pltpu.make_async_copy(
                bufs.at[pl.ds(b * bpw, bpw)],
                acc_sh.at[slot_v.at[b]], ssem.at[b]).wait()

        for b in range(nbuf):
            fill_gidx(b, b)
            gather(b)

        def chunk(c, carry):
            l0 = c * nbuf
            for b in range(nbuf):
                fill_slot(l0 + b, b)
                wait_gather(b)
                scatter_add(b)
                wait_scatter(b)
                fill_gidx(l0 + b + nbuf, b)
                gather(b)
            return carry

        lax.fori_loop(0, nch - 1, chunk, 0)

        l0 = (nch - 1) * nbuf
        for b in range(nbuf):
            fill_slot(l0 + b, b)
            wait_gather(b)
            scatter_add(b)
        for b in range(nbuf):
            wait_scatter(b)

        pltpu.sync_copy(acc_sh.at[pl.ds(s2, bpw)], out_hbm.at[pl.ds(base, bpw)])

    return pool(x, tab)


def _mlp(s0, s1, W1, b1, w2, b2):
    B = s0.shape[0]

    def body(s0_ref, s1_ref, w1_ref, b1_ref, w2_ref, b2_ref, o_ref):
        m = (s0_ref[...] + s1_ref[...]) * (1.0 / _L)
        h = jnp.dot(m, w1_ref[...], preferred_element_type=jnp.float32)
        h = jnp.maximum(h + b1_ref[...][None, :], 0.0)
        z = jnp.sum(h * w2_ref[...][None, :], axis=-1) + b2_ref[0, 0]
        o_ref[...] = (1.0 / (1.0 + jnp.exp(-z)))[:, None]

    return pl.pallas_call(
        body,
        out_shape=jax.ShapeDtypeStruct((B, 1), jnp.float32),
    )(s0, s1, W1, b1, w2, b2)


def kernel(x, table, W1, b1, W2, b2):
    x = x.astype(jnp.int32)
    B = x.shape[1]
    vhalf = table.shape[0] // 2
    s0 = _sc_pool(x, table[:vhalf], 0)
    s1 = _sc_pool(x, table[vhalf:], vhalf)
    out = _mlp(s0, s1, W1, b1, W2.reshape(_D), b2.reshape(1, 1))
    return out.reshape(B)


# speedup vs baseline: 15.9244x; 15.9244x over previous
import functools

import jax
import jax.numpy as jnp
from jax import lax
from jax.experimental import pallas as pl
from jax.experimental.pallas import tpu as pltpu
from jax.experimental.pallas import tpu_sc as plsc

_L = 200
_D = 64

_info = plsc.get_sparse_core_info()
_NC = _info.num_cores
_NS = _info.num_subcores
_NW = _NC * _NS


def _sc_pool(x, table):
    B = x.shape[1]
    bpw = B // _NW
    bpc = bpw * _NS

    mesh = plsc.VectorSubcoreMesh(core_axis_name="c", subcore_axis_name="s")
    nbuf = 8
    nch = _L // nbuf

    @functools.partial(
        pl.kernel,
        mesh=mesh,
        out_type=jax.ShapeDtypeStruct((B, _D), jnp.float32),
        scratch_types=[
            pltpu.VMEM((_L, bpw), jnp.int32),
            pltpu.VMEM((nbuf, bpw, _D), jnp.float32),
            pltpu.VMEM((bpw,), jnp.int32),
            pltpu.VMEM_SHARED((bpc, _D), jnp.float32),
            pltpu.SemaphoreType.DMA((nbuf,)),
            pltpu.SemaphoreType.DMA((nbuf,)),
        ],
        compiler_params=pltpu.CompilerParams(use_tc_tiling_on_sc=False),
    )
    def pool(x_hbm, table_hbm, out_hbm, idx_v, bufs, slot_v, acc_sh, gsem, ssem):
        cid = lax.axis_index("c")
        sid = lax.axis_index("s")
        base = (sid * _NC + cid) * bpw
        sbase = sid * bpw

        zvec = jnp.zeros((16,), jnp.float32)
        for k in range(bpw // 16):
            slot_v[pl.ds(k * 16, 16)] = sbase + k * 16 + lax.iota(jnp.int32, 16)

        def zero_row(i, carry):
            for d in range(_D // 16):
                bufs[0, i, pl.ds(d * 16, 16)] = zvec
            return carry
        lax.fori_loop(0, bpw, zero_row, 0)
        pltpu.sync_copy(bufs.at[0], acc_sh.at[pl.ds(sbase, bpw)])

        pltpu.sync_copy(x_hbm.at[:, pl.ds(base, bpw)], idx_v)

        def gather(l, b):
            return pltpu.async_copy(
                table_hbm.at[idx_v.at[l]], bufs.at[b], gsem.at[b])

        def scatter_add(b):
            return pltpu.async_copy(
                bufs.at[b], acc_sh.at[slot_v], ssem.at[b], add=True)

        for b in range(nbuf):
            gather(b, b)

        def chunk(c, carry):
            l0 = c * nbuf
            for b in range(nbuf):
                pltpu.make_async_copy(
                    table_hbm.at[idx_v.at[l0 + b]], bufs.at[b], gsem.at[b]).wait()
                scatter_add(b)
                pltpu.make_async_copy(
                    bufs.at[b], acc_sh.at[slot_v], ssem.at[b]).wait()
                gather(l0 + b + nbuf, b)
            return carry

        lax.fori_loop(0, nch - 1, chunk, 0)

        l0 = (nch - 1) * nbuf
        for b in range(nbuf):
            pltpu.make_async_copy(
                table_hbm.at[idx_v.at[l0 + b]], bufs.at[b], gsem.at[b]).wait()
            scatter_add(b)
        for b in range(nbuf):
            pltpu.make_async_copy(
                bufs.at[b], acc_sh.at[slot_v], ssem.at[b]).wait()

        pltpu.sync_copy(acc_sh.at[pl.ds(sbase, bpw)], out_hbm.at[pl.ds(base, bpw)])

    return pool(x, table)


def _mlp(sums, W1, b1, w2, b2):
    B = sums.shape[0]

    def body(s_ref, w1_ref, b1_ref, w2_ref, b2_ref, o_ref):
        m = s_ref[...] * (1.0 / _L)
        h = jnp.dot(m, w1_ref[...], preferred_element_type=jnp.float32)
        h = jnp.maximum(h + b1_ref[...][None, :], 0.0)
        z = jnp.sum(h * w2_ref[...][None, :], axis=-1) + b2_ref[0, 0]
        o_ref[...] = (1.0 / (1.0 + jnp.exp(-z)))[:, None]

    return pl.pallas_call(
        body,
        out_shape=jax.ShapeDtypeStruct((B, 1), jnp.float32),
    )(sums, W1, b1, w2, b2)


def kernel(x, table, W1, b1, W2, b2):
    x = x.astype(jnp.int32)
    B = x.shape[1]
    one = (b1[0] + 1.0) - b1[0]
    sums = _sc_pool(x, table * one)
    out = _mlp(sums, W1, b1, W2.reshape(_D), b2.reshape(1, 1))
    return out.reshape(B)


# speedup vs baseline: 16.4806x; 1.0349x over previous
import functools

import jax
import jax.numpy as jnp
from jax import lax
from jax.experimental import pallas as pl
from jax.experimental.pallas import tpu as pltpu
from jax.experimental.pallas import tpu_sc as plsc

_L = 200
_D = 64

_info = plsc.get_sparse_core_info()
_NC = _info.num_cores
_NS = _info.num_subcores
_NW = _NC * _NS


def _sc_pool(x, tab2):
    Lh = x.shape[0]
    B = x.shape[1]
    bpw = B // _NW
    bpc = bpw * _NS

    mesh = plsc.VectorSubcoreMesh(core_axis_name="c", subcore_axis_name="s")
    nbuf = 4
    nch = Lh // nbuf

    @functools.partial(
        pl.kernel,
        mesh=mesh,
        out_type=jax.ShapeDtypeStruct((B // 2, 2 * _D), jnp.float32),
        scratch_types=[
            pltpu.VMEM((Lh, bpw), jnp.int32),
            pltpu.VMEM((nbuf, bpw), jnp.int32),
            pltpu.VMEM((nbuf, bpw), jnp.int32),
            pltpu.VMEM((nbuf * bpw, 2 * _D), jnp.float32),
            pltpu.VMEM((bpw // 2, 2 * _D), jnp.float32),
            pltpu.VMEM_SHARED((2 * bpc, 2 * _D), jnp.float32),
            pltpu.SemaphoreType.DMA((nbuf,)),
            pltpu.SemaphoreType.DMA((nbuf,)),
        ],
    )
    def pool(x_hbm, tab_hbm, out_hbm, idx_v, gidx_v, slot_v, bufs, stage_v,
             acc_sh, gsem, ssem):
        cid = lax.axis_index("c")
        sid = lax.axis_index("s")
        base = pl.multiple_of((sid * _NC + cid) * bpw, bpw)
        bh = pl.multiple_of((sid * _NC + cid) * (bpw // 2), bpw // 2)
        s2 = pl.multiple_of(2 * sid * bpw, 2 * bpw)

        zvec = jnp.zeros((16,), jnp.float32)
        iot = lax.iota(jnp.int32, 16)

        def zero_row(i, carry):
            for d in range(2 * _D // 16):
                bufs[i, pl.ds(d * 16, 16)] = zvec
            return carry
        lax.fori_loop(0, 2 * bpw, zero_row, 0)
        pltpu.sync_copy(bufs.at[pl.ds(0, 2 * bpw)],
                        acc_sh.at[pl.ds(s2, 2 * bpw)])

        pltpu.sync_copy(x_hbm.at[:, pl.ds(base, bpw)], idx_v)

        def fill_gidx(l, b):
            for k in range(bpw // 16):
                v = idx_v[l, pl.ds(k * 16, 16)]
                gidx_v[b, pl.ds(k * 16, 16)] = lax.shift_right_logical(v, 1)

        def fill_slot(l, b):
            for k in range(bpw // 16):
                v = idx_v[l, pl.ds(k * 16, 16)]
                slot_v[b, pl.ds(k * 16, 16)] = (
                    s2 + 2 * (k * 16 + iot) + (v & 1))

        def gather(b):
            return pltpu.async_copy(
                tab_hbm.at[gidx_v.at[b]],
                bufs.at[pl.ds(b * bpw, bpw)], gsem.at[b])

        def scatter_add(b):
            return pltpu.async_copy(
                bufs.at[pl.ds(b * bpw, bpw)],
                acc_sh.at[slot_v.at[b]], ssem.at[b], add=True)

        def wait_gather(b):
            pltpu.make_async_copy(
                tab_hbm.at[gidx_v.at[b]],
                bufs.at[pl.ds(b * bpw, bpw)], gsem.at[b]).wait()

        def wait_scatter(b):
            pltpu.make_async_copy(
                bufs.at[pl.ds(b * bpw, bpw)],
                acc_sh.at[slot_v.at[b]], ssem.at[b]).wait()

        for b in range(nbuf):
            fill_gidx(b, b)
            gather(b)

        def chunk(c, carry):
            l0 = c * nbuf
            for b in range(nbuf):
                fill_slot(l0 + b, b)
                wait_gather(b)
                scatter_add(b)
                wait_scatter(b)
                fill_gidx(l0 + b + nbuf, b)
                gather(b)
            return carry

        lax.fori_loop(0, nch - 1, chunk, 0)

        l0 = (nch - 1) * nbuf
        for b in range(nbuf):
            fill_slot(l0 + b, b)
            wait_gather(b)
            scatter_add(b)
        for b in range(nbuf):
            wait_scatter(b)

        pltpu.sync_copy(acc_sh.at[pl.ds(s2, 2 * bpw)],
                        bufs.at[pl.ds(0, 2 * bpw)])
        def combine(q, carry):
            r = 4 * q
            for d in range(_D // 16):
                stage_v[q, pl.ds(d * 16, 16)] = (
                    bufs[r, pl.ds(d * 16, 16)]
                    + bufs[r + 1, pl.ds(_D + d * 16, 16)])
                stage_v[q, pl.ds(_D + d * 16, 16)] = (
                    bufs[r + 2, pl.ds(d * 16, 16)]
                    + bufs[r + 3, pl.ds(_D + d * 16, 16)])
            return carry
        lax.fori_loop(0, bpw // 2, combine, 0)

        pltpu.sync_copy(stage_v, out_hbm.at[pl.ds(bh, bpw // 2)])

    return pool(x, tab2)


def _mlp(s0, s1, W1, b1, w2, b2):
    B = s0.shape[0]

    def body(s0_ref, s1_ref, w1_ref, b1_ref, w2_ref, b2_ref, o_ref):
        m = (s0_ref[...] + s1_ref[...]) * (1.0 / _L)
        h = jnp.dot(m, w1_ref[...], preferred_element_type=jnp.float32)
        h = jnp.maximum(h + b1_ref[...][None, :], 0.0)
        z = jnp.sum(h * w2_ref[...][None, :], axis=-1) + b2_ref[0, 0]
        o_ref[...] = (1.0 / (1.0 + jnp.exp(-z)))[:, None]

    return pl.pallas_call(
        body,
        out_shape=jax.ShapeDtypeStruct((B, 1), jnp.float32),
    )(s0, s1, W1, b1, w2, b2)


def kernel(x, table, W1, b1, W2, b2):
    x = x.astype(jnp.int32)
    B = x.shape[1]
    one = (b1[0] + 1.0) - b1[0]
    tab2 = table.reshape(table.shape[0] // 2, 2 * _D) * one
    lh = x.shape[0] // 2
    s0 = _sc_pool(x[:lh], tab2).reshape(B, _D)
    s1 = _sc_pool(x[lh:], tab2).reshape(B, _D)
    out = _mlp(s0, s1, W1, b1, W2.reshape(_D), b2.reshape(1, 1))
    return out.reshape(B)


# speedup vs baseline: 22.8361x; 1.3856x over previous
import functools

import jax
import jax.numpy as jnp
from jax import lax
from jax.experimental import pallas as pl
from jax.experimental.pallas import tpu as pltpu
from jax.experimental.pallas import tpu_sc as plsc

_L = 200
_D = 64

_info = plsc.get_sparse_core_info()
_NC = _info.num_cores
_NS = _info.num_subcores
_NW = _NC * _NS


def _sc_pool(x, table):
    B = x.shape[1]
    bpw = B // _NW
    bpc = bpw * _NS

    mesh = plsc.VectorSubcoreMesh(core_axis_name="c", subcore_axis_name="s")
    nbuf = 8
    nch = _L // nbuf

    @functools.partial(
        pl.kernel,
        mesh=mesh,
        out_type=jax.ShapeDtypeStruct((B, _D), jnp.float32),
        scratch_types=[
            pltpu.VMEM((_L, bpw), jnp.int32),
            pltpu.VMEM((nbuf, bpw, _D), jnp.float32),
            pltpu.VMEM((bpw,), jnp.int32),
            pltpu.VMEM_SHARED((bpc, _D), jnp.float32),
            pltpu.SemaphoreType.DMA((nbuf,)),
            pltpu.SemaphoreType.DMA((nbuf,)),
        ],
        compiler_params=pltpu.CompilerParams(use_tc_tiling_on_sc=False),
    )
    def pool(x_hbm, table_hbm, out_hbm, idx_v, bufs, slot_v, acc_sh, gsem, ssem):
        cid = lax.axis_index("c")
        sid = lax.axis_index("s")
        base = (sid * _NC + cid) * bpw
        sbase = sid * bpw

        zvec = jnp.zeros((16,), jnp.float32)
        for k in range(bpw // 16):
            slot_v[pl.ds(k * 16, 16)] = sbase + k * 16 + lax.iota(jnp.int32, 16)

        def zero_row(i, carry):
            for d in range(_D // 16):
                bufs[0, i, pl.ds(d * 16, 16)] = zvec
            return carry
        lax.fori_loop(0, bpw, zero_row, 0)
        pltpu.sync_copy(bufs.at[0], acc_sh.at[pl.ds(sbase, bpw)])

        pltpu.sync_copy(x_hbm.at[:, pl.ds(base, bpw)], idx_v)

        def gather(l, b):
            return pltpu.async_copy(
                table_hbm.at[idx_v.at[l]], bufs.at[b], gsem.at[b])

        def scatter_add(b):
            return pltpu.async_copy(
                bufs.at[b], acc_sh.at[slot_v], ssem.at[b], add=True)

        for b in range(nbuf):
            gather(b, b)

        def chunk(c, carry):
            l0 = c * nbuf
            for b in range(nbuf):
                pltpu.make_async_copy(
                    table_hbm.at[idx_v.at[l0 + b]], bufs.at[b], gsem.at[b]).wait()
                scatter_add(b)
                pltpu.make_async_copy(
                    bufs.at[b], acc_sh.at[slot_v], ssem.at[b]).wait()
                gather(l0 + b + nbuf, b)
            return carry

        lax.fori_loop(0, nch - 1, chunk, 0)

        l0 = (nch - 1) * nbuf
        for b in range(nbuf):
            pltpu.make_async_copy(
                table_hbm.at[idx_v.at[l0 + b]], bufs.at[b], gsem.at[b]).wait()
            scatter_add(b)
        for b in range(nbuf):
            pltpu.make_async_copy(
                bufs.at[b], acc_sh.at[slot_v], ssem.at[b]).wait()

        pltpu.sync_copy(acc_sh.at[pl.ds(sbase, bpw)], out_hbm.at[pl.ds(base, bpw)])

    return pool(x, table)


def _mlp(sums, W1, b1, w2, b2):
    B = sums.shape[0]

    def body(s_ref, w1_ref, b1_ref, w2_ref, b2_ref, o_ref):
        m = s_ref[...] * (1.0 / _L)
        h = jnp.dot(m, w1_ref[...], preferred_element_type=jnp.float32)
        h = jnp.maximum(h + b1_ref[...][None, :], 0.0)
        z = jnp.sum(h * w2_ref[...][None, :], axis=-1) + b2_ref[0, 0]
        o_ref[...] = (1.0 / (1.0 + jnp.exp(-z)))[:, None]

    return pl.pallas_call(
        body,
        out_shape=jax.ShapeDtypeStruct((B, 1), jnp.float32),
    )(sums, W1, b1, w2, b2)


def kernel(x, table, W1, b1, W2, b2):
    x = x.astype(jnp.int32)
    B = x.shape[1]
    sums = _sc_pool(x, table)
    out = _mlp(sums, W1, b1, W2.reshape(_D), b2.reshape(1, 1))
    return out.reshape(B)


# speedup vs baseline: 22.8519x; 1.0007x over previous
import functools

import jax
import jax.numpy as jnp
from jax import lax
from jax.experimental import pallas as pl
from jax.experimental.pallas import tpu as pltpu
from jax.experimental.pallas import tpu_sc as plsc

_L = 200
_D = 64

_info = plsc.get_sparse_core_info()
_NC = _info.num_cores
_NS = _info.num_subcores
_NW = _NC * _NS


def _sc_pool(x, table):
    B = x.shape[1]
    bpw = B // _NW
    bpc = bpw * _NS

    mesh = plsc.VectorSubcoreMesh(core_axis_name="c", subcore_axis_name="s")
    nbuf = 10
    nch = _L // nbuf

    @functools.partial(
        pl.kernel,
        mesh=mesh,
        out_type=jax.ShapeDtypeStruct((B, _D), jnp.float32),
        scratch_types=[
            pltpu.VMEM((_L, bpw), jnp.int32),
            pltpu.VMEM((nbuf, bpw, _D), jnp.float32),
            pltpu.VMEM((bpw,), jnp.int32),
            pltpu.VMEM_SHARED((bpc, _D), jnp.float32),
            pltpu.SemaphoreType.DMA((nbuf,)),
            pltpu.SemaphoreType.DMA((nbuf,)),
        ],
        compiler_params=pltpu.CompilerParams(use_tc_tiling_on_sc=False),
    )
    def pool(x_hbm, table_hbm, out_hbm, idx_v, bufs, slot_v, acc_sh, gsem, ssem):
        cid = lax.axis_index("c")
        sid = lax.axis_index("s")
        base = (sid * _NC + cid) * bpw
        sbase = sid * bpw

        zvec = jnp.zeros((16,), jnp.float32)
        for k in range(bpw // 16):
            slot_v[pl.ds(k * 16, 16)] = sbase + k * 16 + lax.iota(jnp.int32, 16)

        def zero_row(i, carry):
            for d in range(_D // 16):
                bufs[0, i, pl.ds(d * 16, 16)] = zvec
            return carry
        lax.fori_loop(0, bpw, zero_row, 0)
        pltpu.sync_copy(bufs.at[0], acc_sh.at[pl.ds(sbase, bpw)])

        pltpu.sync_copy(x_hbm.at[:, pl.ds(base, bpw)], idx_v)

        def gather(l, b):
            return pltpu.async_copy(
                table_hbm.at[idx_v.at[l]], bufs.at[b], gsem.at[b])

        def scatter_add(b):
            return pltpu.async_copy(
                bufs.at[b], acc_sh.at[slot_v], ssem.at[b], add=True)

        for b in range(nbuf):
            gather(b, b)

        def chunk(c, carry):
            l0 = c * nbuf
            for b in range(nbuf):
                pltpu.make_async_copy(
                    table_hbm.at[idx_v.at[l0 + b]], bufs.at[b], gsem.at[b]).wait()
                scatter_add(b)
                pltpu.make_async_copy(
                    bufs.at[b], acc_sh.at[slot_v], ssem.at[b]).wait()
                gather(l0 + b + nbuf, b)
            return carry

        lax.fori_loop(0, nch - 1, chunk, 0)

        l0 = (nch - 1) * nbuf
        for b in range(nbuf):
            pltpu.make_async_copy(
                table_hbm.at[idx_v.at[l0 + b]], bufs.at[b], gsem.at[b]).wait()
            scatter_add(b)
        for b in range(nbuf):
            pltpu.make_async_copy(
                bufs.at[b], acc_sh.at[slot_v], ssem.at[b]).wait()

        pltpu.sync_copy(acc_sh.at[pl.ds(sbase, bpw)], out_hbm.at[pl.ds(base, bpw)])

    return pool(x, table)


def _mlp(sums, W1, b1, w2, b2):
    B = sums.shape[0]

    def body(s_ref, w1_ref, b1_ref, w2_ref, b2_ref, o_ref):
        m = s_ref[...] * (1.0 / _L)
        h = jnp.dot(m, w1_ref[...], preferred_element_type=jnp.float32)
        h = jnp.maximum(h + b1_ref[...][None, :], 0.0)
        z = jnp.sum(h * w2_ref[...][None, :], axis=-1) + b2_ref[0, 0]
        o_ref[...] = (1.0 / (1.0 + jnp.exp(-z)))[:, None]

    return pl.pallas_call(
        body,
        out_shape=jax.ShapeDtypeStruct((B, 1), jnp.float32),
    )(sums, W1, b1, w2, b2)


def kernel(x, table, W1, b1, W2, b2):
    x = x.astype(jnp.int32)
    B = x.shape[1]
    sums = _sc_pool(x, table)
    out = _mlp(sums, W1, b1, W2.reshape(_D), b2.reshape(1, 1))
    return out.reshape(B)
